# baseline (speedup 1.0000x reference)
import functools

import jax
import jax.numpy as jnp
import numpy as np
from jax import lax
from jax.experimental import pallas as pl
from jax.experimental.pallas import tpu as pltpu
from jax.experimental.pallas import tpu_sc as plsc

_NC = 2
_NS = 16
_NW = _NC * _NS

_CHUNK = 80
_K = 5
_BATCH = _CHUNK * _K

_BE = 2560


def _sc_gather(src_tbl, dst_tbl, src_idx, dst_idx):
  e = src_idx.shape[0]
  d = src_tbl.shape[1]
  epw = e // _NW
  n_batches = epw // _BATCH
  half = _BATCH // 2
  mesh = plsc.VectorSubcoreMesh(core_axis_name="c", subcore_axis_name="s")

  @functools.partial(
      pl.kernel,
      out_type=(jax.ShapeDtypeStruct((e // 2, d), jnp.int32),
                jax.ShapeDtypeStruct((e // 2, d), jnp.int32)),
      mesh=mesh,
      scratch_types=[
          pltpu.VMEM((epw,), jnp.int32),
          pltpu.VMEM((epw,), jnp.int32),
          pltpu.VMEM((_BATCH, d), jnp.float32),
          pltpu.VMEM((half, d), jnp.int32),
          pltpu.VMEM((half, d), jnp.int32),
          pltpu.SemaphoreType.DMA,
          pltpu.SemaphoreType.DMA,
          pltpu.SemaphoreType.DMA,
      ],
      compiler_params=pltpu.CompilerParams(needs_layout_passes=False),
  )
  def gather_kernel(src_tbl_hbm, dst_tbl_hbm, sidx_hbm, didx_hbm,
                    src_out, dst_out, sidx_v, didx_v, sbuf,
                    osbuf, odbuf, gsem, wsem_s, wsem_d):
    wid = lax.axis_index("s") * _NC + lax.axis_index("c")
    base = wid * epw
    obase = wid * (epw // 2)
    pltpu.sync_copy(sidx_hbm.at[pl.ds(base, epw)], sidx_v)
    pltpu.sync_copy(didx_hbm.at[pl.ds(base, epw)], didx_v)

    def fire_gathers(tbl, idx_v, buf, off):
      copies = []
      for j in range(_K):
        co = off + j * _CHUNK
        copies.append(pltpu.async_copy(
            tbl.at[idx_v.at[pl.ds(co, _CHUNK)]],
            buf.at[pl.ds(j * _CHUNK, _CHUNK)], gsem))
      return copies

    def convert(buf, obuf):
      rnd = jnp.uint32(0x8000)
      hmask = jnp.uint32(0xFFFF0000)

      def rows(r2, carry):
        for g in range(8):
          a = plsc.bitcast(buf[2 * r2, pl.ds(16 * g, 16)], jnp.uint32)
          b = plsc.bitcast(buf[2 * r2 + 1, pl.ds(16 * g, 16)], jnp.uint32)
          w = ((a + rnd) >> 16) | ((b + rnd) & hmask)
          obuf[r2, pl.ds(16 * g, 16)] = plsc.bitcast(w, jnp.int32)
        return carry
      lax.fori_loop(0, half, rows, 0, unroll=4)

    def drain_write(obuf, out, ooff, wsem):
      pltpu.make_async_copy(obuf, out.at[pl.ds(obase + ooff, half)],
                            wsem).wait()

    def body(b, carry):
      off = b * _BATCH
      ooff = b * half
      oprev = (b - 1) * half

      sc = fire_gathers(src_tbl_hbm, sidx_v, sbuf, off)
      for c in sc:
        c.wait()

      @pl.when(b > 0)
      def _():
        drain_write(osbuf, src_out, oprev, wsem_s)

      convert(sbuf, osbuf)
      pltpu.async_copy(osbuf, src_out.at[pl.ds(obase + ooff, half)], wsem_s)
      dc = fire_gathers(dst_tbl_hbm, didx_v, sbuf, off)
      for c in dc:
        c.wait()

      @pl.when(b > 0)
      def _():
        drain_write(odbuf, dst_out, oprev, wsem_d)

      convert(sbuf, odbuf)
      pltpu.async_copy(odbuf, dst_out.at[pl.ds(obase + ooff, half)], wsem_d)
      return carry

    lax.fori_loop(0, n_batches, body, 0)
    olast = (n_batches - 1) * half
    drain_write(osbuf, src_out, olast, wsem_s)
    drain_write(odbuf, dst_out, olast, wsem_d)

  return gather_kernel(src_tbl, dst_tbl, src_idx, dst_idx)


def _mlp_body(s_ref, d_ref, e_ref, w1_ref, b1_ref, w2_ref, b2_ref,
              g_ref, bt_ref, o_ref):
  ef = e_ref[:]
  s = pltpu.bitcast(s_ref[:], jnp.bfloat16)
  d = pltpu.bitcast(d_ref[:], jnp.bfloat16)
  cat = jnp.concatenate([s, d, ef.astype(jnp.bfloat16)], axis=1)
  h = lax.dot_general(cat, w1_ref[:], (((1,), (0,)), ((), ())),
                      preferred_element_type=jnp.float32)
  h += b1_ref[:]
  h = h * (0.5 * jnp.tanh(0.5 * h) + 0.5)
  h2 = lax.dot_general(h.astype(jnp.bfloat16), w2_ref[:],
                       (((1,), (0,)), ((), ())),
                       preferred_element_type=jnp.float32)
  h2 += b2_ref[:]
  mu = jnp.mean(h2, axis=1, keepdims=True)
  xc = h2 - mu
  var = jnp.mean(xc * xc, axis=1, keepdims=True)
  o_ref[:] = xc * lax.rsqrt(var + 1e-5) * g_ref[:] + bt_ref[:] + ef


def _tc_mlp_seg(acc, src_g, dst_g, edge, w1, b1, w2, b2, gamma, beta,
                blk0, seg_e):
  e, d = edge.shape
  h = w1.shape[1]
  nb = seg_e // _BE

  def seg_map(i):
    return (i + blk0, 0)

  def body(acc_ref, s_ref, d_ref, e_ref, w1_ref, b1_ref, w2_ref, b2_ref,
           g_ref, bt_ref, o_ref):
    del acc_ref
    _mlp_body(s_ref, d_ref, e_ref, w1_ref, b1_ref, w2_ref, b2_ref,
              g_ref, bt_ref, o_ref)

  fn = body if acc is not None else _mlp_body
  in_specs = [
      pl.BlockSpec((_BE // 2, d), lambda i: (i, 0)),
      pl.BlockSpec((_BE // 2, d), lambda i: (i, 0)),
      pl.BlockSpec((_BE, d), seg_map),
      pl.BlockSpec((3 * d, h), lambda i: (0, 0)),
      pl.BlockSpec((1, h), lambda i: (0, 0)),
      pl.BlockSpec((h, d), lambda i: (0, 0)),
      pl.BlockSpec((1, d), lambda i: (0, 0)),
      pl.BlockSpec((1, d), lambda i: (0, 0)),
      pl.BlockSpec((1, d), lambda i: (0, 0)),
  ]
  args = [src_g, dst_g, edge,
          w1.astype(jnp.bfloat16), b1.reshape(1, h),
          w2.astype(jnp.bfloat16), b2.reshape(1, d),
          gamma.reshape(1, d), beta.reshape(1, d)]
  kwargs = {}
  if acc is not None:
    in_specs = [pl.BlockSpec(memory_space=pl.ANY)] + in_specs
    args = [acc] + args
    kwargs["input_output_aliases"] = {0: 0}
  return pl.pallas_call(
      fn,
      grid=(nb,),
      in_specs=in_specs,
      out_specs=pl.BlockSpec((_BE, d), seg_map),
      out_shape=jax.ShapeDtypeStruct((e, d), jnp.float32),
      **kwargs,
  )(*args)


_SEG_SIZES = (64000, 64000, 64000, 64000, 64000)


def kernel(src_node_features, dst_node_features, edge_features,
           src_indices, dst_indices, W1, b1, W2, b2, ln_gamma, ln_beta):
  e, d = edge_features.shape
  src_idx = src_indices.astype(jnp.int32)
  dst_idx = dst_indices.astype(jnp.int32)
  gathered = []
  off = 0
  for seg_e in _SEG_SIZES:
    sl = slice(off, off + seg_e)
    gathered.append(_sc_gather(src_node_features, dst_node_features,
                               src_idx[sl], dst_idx[sl]))
    off += seg_e
  acc = None
  off = 0
  for j, seg_e in enumerate(_SEG_SIZES):
    src_g, dst_g = gathered[j]
    acc = _tc_mlp_seg(acc, src_g, dst_g, edge_features,
                      W1, b1, W2, b2, ln_gamma, ln_beta, off // _BE, seg_e)
    off += seg_e
  return acc


# speedup vs baseline: 1.4091x; 1.4091x over previous
import functools

import jax
import jax.numpy as jnp
import numpy as np
from jax import lax
from jax.experimental import pallas as pl
from jax.experimental.pallas import tpu as pltpu
from jax.experimental.pallas import tpu_sc as plsc

_NC = 2
_NS = 16
_NW = _NC * _NS

_CHUNK = 80
_K = 5
_BATCH = _CHUNK * _K

_BE = 2560


def _sc_gather(src_tbl, dst_tbl, src_idx, dst_idx):
  e = src_idx.shape[0]
  d = src_tbl.shape[1]
  epw = e // _NW
  n_batches = epw // _BATCH
  half = _BATCH // 2
  mesh = plsc.VectorSubcoreMesh(core_axis_name="c", subcore_axis_name="s")

  @functools.partial(
      pl.kernel,
      out_type=(jax.ShapeDtypeStruct((e, d), jnp.float32),
                jax.ShapeDtypeStruct((e, d), jnp.float32)),
      mesh=mesh,
      scratch_types=[
          pltpu.VMEM((epw,), jnp.int32),
          pltpu.VMEM((epw,), jnp.int32),
          pltpu.VMEM((_BATCH, d), jnp.float32),
          pltpu.VMEM((_BATCH, d), jnp.float32),
          pltpu.SemaphoreType.DMA,
          pltpu.SemaphoreType.DMA,
          pltpu.SemaphoreType.DMA,
      ],
  )
  def gather_kernel(src_tbl_hbm, dst_tbl_hbm, sidx_hbm, didx_hbm,
                    src_out, dst_out, sidx_v, didx_v, sbuf, dbuf,
                    gsem, wsem_s, wsem_d):
    wid = lax.axis_index("s") * _NC + lax.axis_index("c")
    base = wid * epw
    pltpu.sync_copy(sidx_hbm.at[pl.ds(base, epw)], sidx_v)
    pltpu.sync_copy(didx_hbm.at[pl.ds(base, epw)], didx_v)

    def fire_gathers(tbl, idx_v, buf, off):
      copies = []
      for j in range(_K):
        co = off + j * _CHUNK
        copies.append(pltpu.async_copy(
            tbl.at[idx_v.at[pl.ds(co, _CHUNK)]],
            buf.at[pl.ds(j * _CHUNK, _CHUNK)], gsem))
      return copies

    def drain_write(buf, out, off, wsem):
      pltpu.make_async_copy(buf, out.at[pl.ds(base + off, _BATCH)],
                            wsem).wait()

    def body(b, carry):
      off = b * _BATCH
      prev = (b - 1) * _BATCH

      @pl.when(b > 0)
      def _():
        drain_write(sbuf, src_out, prev, wsem_s)

      sc = fire_gathers(src_tbl_hbm, sidx_v, sbuf, off)

      @pl.when(b > 0)
      def _():
        drain_write(dbuf, dst_out, prev, wsem_d)

      dc = fire_gathers(dst_tbl_hbm, didx_v, dbuf, off)
      for c in sc:
        c.wait()
      pltpu.async_copy(sbuf, src_out.at[pl.ds(base + off, _BATCH)], wsem_s)
      for c in dc:
        c.wait()
      pltpu.async_copy(dbuf, dst_out.at[pl.ds(base + off, _BATCH)], wsem_d)
      return carry

    lax.fori_loop(0, n_batches, body, 0)
    last = (n_batches - 1) * _BATCH
    drain_write(sbuf, src_out, last, wsem_s)
    drain_write(dbuf, dst_out, last, wsem_d)

  return gather_kernel(src_tbl, dst_tbl, src_idx, dst_idx)


def _mlp_body(s_ref, d_ref, e_ref, w1_ref, b1_ref, w2_ref, b2_ref,
              g_ref, bt_ref, o_ref):
  ef = e_ref[:]
  cat = jnp.concatenate(
      [s_ref[:].astype(jnp.bfloat16),
       d_ref[:].astype(jnp.bfloat16),
       ef.astype(jnp.bfloat16)], axis=1)
  h = lax.dot_general(cat, w1_ref[:], (((1,), (0,)), ((), ())),
                      preferred_element_type=jnp.float32)
  h += b1_ref[:]
  h = h * (0.5 * jnp.tanh(0.5 * h) + 0.5)
  h2 = lax.dot_general(h.astype(jnp.bfloat16), w2_ref[:],
                       (((1,), (0,)), ((), ())),
                       preferred_element_type=jnp.float32)
  h2 += b2_ref[:]
  mu = jnp.mean(h2, axis=1, keepdims=True)
  xc = h2 - mu
  var = jnp.mean(xc * xc, axis=1, keepdims=True)
  o_ref[:] = xc * lax.rsqrt(var + 1e-5) * g_ref[:] + bt_ref[:] + ef


def _tc_mlp_seg(acc, src_g, dst_g, edge, w1, b1, w2, b2, gamma, beta,
                blk0, seg_e):
  e, d = edge.shape
  h = w1.shape[1]
  nb = seg_e // _BE

  def seg_map(i):
    return (i + blk0, 0)

  def body(acc_ref, s_ref, d_ref, e_ref, w1_ref, b1_ref, w2_ref, b2_ref,
           g_ref, bt_ref, o_ref):
    del acc_ref
    _mlp_body(s_ref, d_ref, e_ref, w1_ref, b1_ref, w2_ref, b2_ref,
              g_ref, bt_ref, o_ref)

  fn = body if acc is not None else _mlp_body
  in_specs = [
      pl.BlockSpec((_BE, d), lambda i: (i, 0)),
      pl.BlockSpec((_BE, d), lambda i: (i, 0)),
      pl.BlockSpec((_BE, d), seg_map),
      pl.BlockSpec((3 * d, h), lambda i: (0, 0)),
      pl.BlockSpec((1, h), lambda i: (0, 0)),
      pl.BlockSpec((h, d), lambda i: (0, 0)),
      pl.BlockSpec((1, d), lambda i: (0, 0)),
      pl.BlockSpec((1, d), lambda i: (0, 0)),
      pl.BlockSpec((1, d), lambda i: (0, 0)),
  ]
  args = [src_g, dst_g, edge,
          w1.astype(jnp.bfloat16), b1.reshape(1, h),
          w2.astype(jnp.bfloat16), b2.reshape(1, d),
          gamma.reshape(1, d), beta.reshape(1, d)]
  kwargs = {}
  if acc is not None:
    in_specs = [pl.BlockSpec(memory_space=pl.ANY)] + in_specs
    args = [acc] + args
    kwargs["input_output_aliases"] = {0: 0}
  return pl.pallas_call(
      fn,
      grid=(nb,),
      in_specs=in_specs,
      out_specs=pl.BlockSpec((_BE, d), seg_map),
      out_shape=jax.ShapeDtypeStruct((e, d), jnp.float32),
      **kwargs,
  )(*args)


_SEG_SIZES = (64000, 64000, 64000, 64000, 64000)


def kernel(src_node_features, dst_node_features, edge_features,
           src_indices, dst_indices, W1, b1, W2, b2, ln_gamma, ln_beta):
  e, d = edge_features.shape
  src_idx = src_indices.astype(jnp.int32)
  dst_idx = dst_indices.astype(jnp.int32)
  gathered = []
  off = 0
  for seg_e in _SEG_SIZES:
    sl = slice(off, off + seg_e)
    gathered.append(_sc_gather(src_node_features, dst_node_features,
                               src_idx[sl], dst_idx[sl]))
    off += seg_e
  acc = None
  off = 0
  for j, seg_e in enumerate(_SEG_SIZES):
    src_g, dst_g = gathered[j]
    acc = _tc_mlp_seg(acc, src_g, dst_g, edge_features,
                      W1, b1, W2, b2, ln_gamma, ln_beta, off // _BE, seg_e)
    off += seg_e
  return acc


# speedup vs baseline: 1.4250x; 1.0113x over previous
import functools

import jax
import jax.numpy as jnp
import numpy as np
from jax import lax
from jax.experimental import pallas as pl
from jax.experimental.pallas import tpu as pltpu
from jax.experimental.pallas import tpu_sc as plsc

_NC = 2
_NS = 16
_NW = _NC * _NS

_CHUNK = 80
_K = 5
_BATCH = _CHUNK * _K

_BE = 3200


def _sc_gather(src_tbl, dst_tbl, src_idx, dst_idx):
  e = src_idx.shape[0]
  d = src_tbl.shape[1]
  epw = e // _NW
  n_batches = epw // _BATCH
  half = _BATCH // 2
  mesh = plsc.VectorSubcoreMesh(core_axis_name="c", subcore_axis_name="s")

  @functools.partial(
      pl.kernel,
      out_type=(jax.ShapeDtypeStruct((e, d), jnp.float32),
                jax.ShapeDtypeStruct((e, d), jnp.float32)),
      mesh=mesh,
      scratch_types=[
          pltpu.VMEM((epw,), jnp.int32),
          pltpu.VMEM((epw,), jnp.int32),
          pltpu.VMEM((_BATCH, d), jnp.float32),
          pltpu.VMEM((_BATCH, d), jnp.float32),
          pltpu.SemaphoreType.DMA,
          pltpu.SemaphoreType.DMA,
          pltpu.SemaphoreType.DMA,
      ],
  )
  def gather_kernel(src_tbl_hbm, dst_tbl_hbm, sidx_hbm, didx_hbm,
                    src_out, dst_out, sidx_v, didx_v, sbuf, dbuf,
                    gsem, wsem_s, wsem_d):
    wid = lax.axis_index("s") * _NC + lax.axis_index("c")
    base = wid * epw
    pltpu.sync_copy(sidx_hbm.at[pl.ds(base, epw)], sidx_v)
    pltpu.sync_copy(didx_hbm.at[pl.ds(base, epw)], didx_v)

    def fire_gathers(tbl, idx_v, buf, off):
      copies = []
      for j in range(_K):
        co = off + j * _CHUNK
        copies.append(pltpu.async_copy(
            tbl.at[idx_v.at[pl.ds(co, _CHUNK)]],
            buf.at[pl.ds(j * _CHUNK, _CHUNK)], gsem))
      return copies

    def drain_write(buf, out, off, wsem):
      pltpu.make_async_copy(buf, out.at[pl.ds(base + off, _BATCH)],
                            wsem).wait()

    def body(b, carry):
      off = b * _BATCH
      prev = (b - 1) * _BATCH

      @pl.when(b > 0)
      def _():
        drain_write(sbuf, src_out, prev, wsem_s)

      sc = fire_gathers(src_tbl_hbm, sidx_v, sbuf, off)

      @pl.when(b > 0)
      def _():
        drain_write(dbuf, dst_out, prev, wsem_d)

      dc = fire_gathers(dst_tbl_hbm, didx_v, dbuf, off)
      for c in sc:
        c.wait()
      pltpu.async_copy(sbuf, src_out.at[pl.ds(base + off, _BATCH)], wsem_s)
      for c in dc:
        c.wait()
      pltpu.async_copy(dbuf, dst_out.at[pl.ds(base + off, _BATCH)], wsem_d)
      return carry

    lax.fori_loop(0, n_batches, body, 0)
    last = (n_batches - 1) * _BATCH
    drain_write(sbuf, src_out, last, wsem_s)
    drain_write(dbuf, dst_out, last, wsem_d)

  return gather_kernel(src_tbl, dst_tbl, src_idx, dst_idx)


def _mlp_body(s_ref, d_ref, e_ref, w1_ref, b1_ref, w2_ref, b2_ref,
              g_ref, bt_ref, o_ref):
  ef = e_ref[:]
  cat = jnp.concatenate(
      [s_ref[:].astype(jnp.bfloat16),
       d_ref[:].astype(jnp.bfloat16),
       ef.astype(jnp.bfloat16)], axis=1)
  h = lax.dot_general(cat, w1_ref[:], (((1,), (0,)), ((), ())),
                      preferred_element_type=jnp.float32)
  h += b1_ref[:]
  h = h * (0.5 * jnp.tanh(0.5 * h) + 0.5)
  h2 = lax.dot_general(h.astype(jnp.bfloat16), w2_ref[:],
                       (((1,), (0,)), ((), ())),
                       preferred_element_type=jnp.float32)
  h2 += b2_ref[:]
  mu = jnp.mean(h2, axis=1, keepdims=True)
  xc = h2 - mu
  var = jnp.mean(xc * xc, axis=1, keepdims=True)
  o_ref[:] = xc * lax.rsqrt(var + 1e-5) * g_ref[:] + bt_ref[:] + ef


def _tc_mlp_seg(acc, src_g, dst_g, edge, w1, b1, w2, b2, gamma, beta,
                blk0, seg_e):
  e, d = edge.shape
  h = w1.shape[1]
  nb = seg_e // _BE

  def seg_map(i):
    return (i + blk0, 0)

  def body(acc_ref, s_ref, d_ref, e_ref, w1_ref, b1_ref, w2_ref, b2_ref,
           g_ref, bt_ref, o_ref):
    del acc_ref
    _mlp_body(s_ref, d_ref, e_ref, w1_ref, b1_ref, w2_ref, b2_ref,
              g_ref, bt_ref, o_ref)

  fn = body if acc is not None else _mlp_body
  in_specs = [
      pl.BlockSpec((_BE, d), lambda i: (i, 0)),
      pl.BlockSpec((_BE, d), lambda i: (i, 0)),
      pl.BlockSpec((_BE, d), seg_map),
      pl.BlockSpec((3 * d, h), lambda i: (0, 0)),
      pl.BlockSpec((1, h), lambda i: (0, 0)),
      pl.BlockSpec((h, d), lambda i: (0, 0)),
      pl.BlockSpec((1, d), lambda i: (0, 0)),
      pl.BlockSpec((1, d), lambda i: (0, 0)),
      pl.BlockSpec((1, d), lambda i: (0, 0)),
  ]
  args = [src_g, dst_g, edge,
          w1.astype(jnp.bfloat16), b1.reshape(1, h),
          w2.astype(jnp.bfloat16), b2.reshape(1, d),
          gamma.reshape(1, d), beta.reshape(1, d)]
  kwargs = {}
  if acc is not None:
    in_specs = [pl.BlockSpec(memory_space=pl.ANY)] + in_specs
    args = [acc] + args
    kwargs["input_output_aliases"] = {0: 0}
  return pl.pallas_call(
      fn,
      grid=(nb,),
      in_specs=in_specs,
      out_specs=pl.BlockSpec((_BE, d), seg_map),
      out_shape=jax.ShapeDtypeStruct((e, d), jnp.float32),
      **kwargs,
  )(*args)


_SEG_SIZES = (12800, 76800, 76800, 76800, 76800)


def kernel(src_node_features, dst_node_features, edge_features,
           src_indices, dst_indices, W1, b1, W2, b2, ln_gamma, ln_beta):
  e, d = edge_features.shape
  src_idx = src_indices.astype(jnp.int32)
  dst_idx = dst_indices.astype(jnp.int32)
  gathered = []
  off = 0
  for seg_e in _SEG_SIZES:
    sl = slice(off, off + seg_e)
    gathered.append(_sc_gather(src_node_features, dst_node_features,
                               src_idx[sl], dst_idx[sl]))
    off += seg_e
  acc = None
  off = 0
  for j, seg_e in enumerate(_SEG_SIZES):
    src_g, dst_g = gathered[j]
    acc = _tc_mlp_seg(acc, src_g, dst_g, edge_features,
                      W1, b1, W2, b2, ln_gamma, ln_beta, off // _BE, seg_e)
    off += seg_e
  return acc
